# baseline (speedup 1.0000x reference)
import jax
import jax.numpy as jnp
from jax.experimental import pallas as pl
from jax.experimental.pallas import tpu as pltpu


def _sigmoid_block(x_ref, o_ref):
    o_ref[...] = 0.5 * jnp.tanh(0.5 * x_ref[...]) + 0.5


def kernel(x):
    batch, nodes = x.shape
    block_rows = 512
    return pl.pallas_call(
        _sigmoid_block,
        out_shape=jax.ShapeDtypeStruct(x.shape, x.dtype),
        grid=(batch // block_rows,),
        in_specs=[pl.BlockSpec((block_rows, nodes), lambda i: (i, 0))],
        out_specs=pl.BlockSpec((block_rows, nodes), lambda i: (i, 0)),
        compiler_params=pltpu.CompilerParams(
            dimension_semantics=("parallel",)),
    )(x)


# speedup vs baseline: 4.0632x; 4.0632x over previous
import jax
import jax.numpy as jnp
from jax.experimental import pallas as pl
from jax.experimental.pallas import tpu as pltpu

_ROWS = 40
_DEPTH = 8


def _sigmoid_stream(x_hbm, o_hbm, in_buf, out_buf, load_sems, store_sems):
    nchunks = x_hbm.shape[0] // _ROWS

    def load(i, slot):
        return pltpu.make_async_copy(
            x_hbm.at[pl.ds(i * _ROWS, _ROWS), :], in_buf.at[slot],
            load_sems.at[slot])

    def store(i, slot):
        return pltpu.make_async_copy(
            out_buf.at[slot], o_hbm.at[pl.ds(i * _ROWS, _ROWS), :],
            store_sems.at[slot])

    for k in range(min(_DEPTH, nchunks)):
        load(k, k).start()

    for i in range(nchunks):
        slot = i % _DEPTH
        load(i, slot).wait()
        if i >= _DEPTH:
            store(i - _DEPTH, slot).wait()
        out_buf[slot] = 0.5 * jnp.tanh(0.5 * in_buf[slot]) + 0.5
        store(i, slot).start()
        if i + _DEPTH < nchunks:
            load(i + _DEPTH, slot).start()

    for i in range(max(nchunks - _DEPTH, 0), nchunks):
        store(i, i % _DEPTH).wait()


def kernel(x):
    xt = x.T
    rows, cols = xt.shape
    out_t = pl.pallas_call(
        _sigmoid_stream,
        out_shape=jax.ShapeDtypeStruct((rows, cols), x.dtype),
        in_specs=[pl.BlockSpec(memory_space=pl.ANY)],
        out_specs=pl.BlockSpec(memory_space=pl.ANY),
        scratch_shapes=[
            pltpu.VMEM((_DEPTH, _ROWS, cols), x.dtype),
            pltpu.VMEM((_DEPTH, _ROWS, cols), x.dtype),
            pltpu.SemaphoreType.DMA((_DEPTH,)),
            pltpu.SemaphoreType.DMA((_DEPTH,)),
        ],
    )(xt)
    return out_t.T
